# Initial kernel scaffold; baseline (speedup 1.0000x reference)
import jax
import jax.numpy as jnp
from jax.experimental import pallas as pl


def kernel(input_seq, table):
    raise NotImplementedError("write your pallas kernel here")


# speedup vs baseline: 2.9739x; 2.9739x over previous
import functools

import jax
import jax.numpy as jnp
from jax import lax
from jax.experimental import pallas as pl
from jax.experimental.pallas import tpu as pltpu
from jax.experimental.pallas import tpu_sc as plsc

_INFO = plsc.get_sparse_core_info()
_NC, _NS = _INFO.num_cores, _INFO.num_subcores
_NW = _NC * _NS

_CHUNK = 128
_B = 4096 * 50
_NCHUNKS = _B // _CHUNK
_CPW = _NCHUNKS // _NW
_EMB = 128


def _embed_body(table_hbm, idx_hbm, out_hbm, idx_v, rows_v, sem):
    wid = lax.axis_index("s") * _NC + lax.axis_index("c")
    pltpu.sync_copy(idx_hbm.at[wid], idx_v)

    def step(j, carry):
        pltpu.async_copy(table_hbm.at[idx_v.at[j]], rows_v, sem).wait()
        pltpu.sync_copy(
            rows_v, out_hbm.at[pl.ds((wid * _CPW + j) * _CHUNK, _CHUNK)]
        )
        return carry

    lax.fori_loop(0, _CPW, step, 0)


@jax.jit
def _embed(idx2d, table):
    mesh = plsc.VectorSubcoreMesh(core_axis_name="c", subcore_axis_name="s")
    fn = pl.kernel(
        _embed_body,
        mesh=mesh,
        out_type=jax.ShapeDtypeStruct((_B, _EMB), jnp.float32),
        scratch_types=[
            pltpu.VMEM((_CPW, _CHUNK), jnp.int32),
            pltpu.VMEM((_CHUNK, _EMB), jnp.float32),
            pltpu.SemaphoreType.DMA,
        ],
    )
    return fn(table, idx2d)


def kernel(input_seq, table):
    idx2d = input_seq.astype(jnp.int32).reshape(_NW, _CPW, _CHUNK)
    out = _embed(idx2d, table)
    return out.reshape(input_seq.shape[0], input_seq.shape[1], _EMB)


# speedup vs baseline: 3.3101x; 1.1130x over previous
import functools

import jax
import jax.numpy as jnp
from jax import lax
from jax.experimental import pallas as pl
from jax.experimental.pallas import tpu as pltpu
from jax.experimental.pallas import tpu_sc as plsc

_INFO = plsc.get_sparse_core_info()
_NC, _NS = _INFO.num_cores, _INFO.num_subcores
_NW = _NC * _NS

_CHUNK = 128
_B = 4096 * 50
_NCHUNKS = _B // _CHUNK
_CPW = _NCHUNKS // _NW
_EMB = 128


_NB = 5
_NGROUPS = _CPW // _NB


def _embed_body(table_hbm, idx_hbm, out_hbm, idx_v, r0, r1, r2, r3, r4,
                gsem, ssem):
    rows = [r0, r1, r2, r3, r4]
    wid = lax.axis_index("s") * _NC + lax.axis_index("c")
    base = wid * _CPW
    pltpu.sync_copy(idx_hbm.at[wid], idx_v)

    for b in range(_NB):
        pltpu.async_copy(table_hbm.at[idx_v.at[b]], rows[b], gsem.at[b])

    def group(g, carry):
        j0 = g * _NB
        for b in range(_NB):
            pltpu.make_async_copy(
                table_hbm.at[idx_v.at[j0 + b]], rows[b], gsem.at[b]
            ).wait()
            pltpu.async_copy(
                rows[b],
                out_hbm.at[pl.ds((base + j0 + b) * _CHUNK, _CHUNK)],
                ssem.at[b],
            )
        for b in range(_NB):
            pltpu.make_async_copy(
                rows[b],
                out_hbm.at[pl.ds((base + j0 + b) * _CHUNK, _CHUNK)],
                ssem.at[b],
            ).wait()
            jn = jnp.minimum(j0 + _NB + b, _CPW - 1)

            @pl.when(g + 1 < _NGROUPS)
            def _():
                pltpu.async_copy(table_hbm.at[idx_v.at[jn]], rows[b],
                                 gsem.at[b])

        return carry

    lax.fori_loop(0, _NGROUPS, group, 0)


@jax.jit
def _embed(idx2d, table):
    mesh = plsc.VectorSubcoreMesh(core_axis_name="c", subcore_axis_name="s")
    fn = pl.kernel(
        _embed_body,
        mesh=mesh,
        out_type=jax.ShapeDtypeStruct((_B, _EMB), jnp.float32),
        scratch_types=[
            pltpu.VMEM((_CPW, _CHUNK), jnp.int32),
        ]
        + [pltpu.VMEM((_CHUNK, _EMB), jnp.float32) for _ in range(_NB)]
        + [
            pltpu.SemaphoreType.DMA((_NB,)),
            pltpu.SemaphoreType.DMA((_NB,)),
        ],
    )
    return fn(table, idx2d)


def kernel(input_seq, table):
    idx2d = input_seq.astype(jnp.int32).reshape(_NW, _CPW, _CHUNK)
    out = _embed(idx2d, table)
    return out.reshape(input_seq.shape[0], input_seq.shape[1], _EMB)


# speedup vs baseline: 5.9314x; 1.7919x over previous
import jax
import jax.numpy as jnp
from jax import lax
from jax.experimental import pallas as pl
from jax.experimental.pallas import tpu as pltpu
from jax.experimental.pallas import tpu_sc as plsc

_INFO = plsc.get_sparse_core_info()
_NC, _NS = _INFO.num_cores, _INFO.num_subcores
_NW = _NC * _NS

_ROWS = 4096
_SEQ = 50
_RPW = _ROWS // _NW
_EMB = 128

_NB = 8
_NGROUPS = _RPW // _NB


def _embed_body(table_hbm, idx_hbm, out_hbm, idx_v, *bufs_and_sems):
    rows = bufs_and_sems[:_NB]
    gsem, ssem = bufs_and_sems[_NB], bufs_and_sems[_NB + 1]
    wid = lax.axis_index("s") * _NC + lax.axis_index("c")
    base = wid * _RPW
    pltpu.sync_copy(idx_hbm.at[pl.ds(base, _RPW)], idx_v)

    for b in range(_NB):
        pltpu.async_copy(table_hbm.at[idx_v.at[b]], rows[b], gsem.at[b])

    def group(g, carry):
        j0 = g * _NB
        for b in range(_NB):
            pltpu.make_async_copy(
                table_hbm.at[idx_v.at[j0 + b]], rows[b], gsem.at[b]
            ).wait()
            pltpu.async_copy(rows[b], out_hbm.at[base + j0 + b], ssem.at[b])
        for b in range(_NB):
            pltpu.make_async_copy(
                rows[b], out_hbm.at[base + j0 + b], ssem.at[b]
            ).wait()
            jn = jnp.minimum(j0 + _NB + b, _RPW - 1)

            @pl.when(g + 1 < _NGROUPS)
            def _():
                pltpu.async_copy(table_hbm.at[idx_v.at[jn]], rows[b],
                                 gsem.at[b])

        return carry

    lax.fori_loop(0, _NGROUPS, group, 0)


@jax.jit
def _embed(input_seq, table):
    mesh = plsc.VectorSubcoreMesh(core_axis_name="c", subcore_axis_name="s")
    fn = pl.kernel(
        _embed_body,
        mesh=mesh,
        out_type=jax.ShapeDtypeStruct((_ROWS, _SEQ, _EMB), jnp.float32),
        scratch_types=[
            pltpu.VMEM((_RPW, _SEQ), jnp.int32),
        ]
        + [pltpu.VMEM((_SEQ, _EMB), jnp.float32) for _ in range(_NB)]
        + [
            pltpu.SemaphoreType.DMA((_NB,)),
            pltpu.SemaphoreType.DMA((_NB,)),
        ],
    )
    return fn(table, input_seq)


def kernel(input_seq, table):
    return _embed(input_seq, table)
